# baseline (speedup 1.0000x reference)
import functools

import jax
import jax.numpy as jnp
from jax import lax
from jax.experimental import pallas as pl
from jax.experimental.pallas import tpu as pltpu
from jax.experimental.pallas import tpu_sc as plsc

N = 10000
H = 128
E = 320000
L = 100000

NC = 2
NS = 16
NW = NC * NS
LANES = 16

C = 128
CE = 64
CW = 160
HALF = CW // 2
NBUF = 2
NP = 10240
RSUB = NP // NS
CL = 26

_f32 = jnp.float32
_i32 = jnp.int32


def _mesh():
    return plsc.VectorSubcoreMesh(core_axis_name="c", subcore_axis_name="s")


def _wid():
    return lax.axis_index("c") * NS + lax.axis_index("s")


def _counts_body(dw_hbm, dr_hbm, outw_hbm, outr_hbm, idx_v, cnt_v):
    wid = _wid()
    ones = jnp.ones((LANES,), _f32)
    zeros = jnp.zeros((LANES,), _f32)

    def one_type(d_hbm, out_hbm):
        def z(i, _):
            cnt_v[pl.ds(i * LANES, LANES)] = zeros
            return 0
        lax.fori_loop(0, NP // LANES, z, 0)
        pltpu.sync_copy(d_hbm.at[wid], idx_v)

        def upd(i, _):
            idx = idx_v[pl.ds(i * LANES, LANES)]
            plsc.addupdate_scatter(cnt_v, [idx], ones)
            return 0
        lax.fori_loop(0, (2 * (HALF + NBUF) * CE) // LANES, upd, 0)
        pltpu.sync_copy(cnt_v, out_hbm.at[wid])

    one_type(dw_hbm, outw_hbm)
    one_type(dr_hbm, outr_hbm)


@jax.jit
def _sc_counts(dw, dr):
    return pl.kernel(
        _counts_body,
        out_type=[
            jax.ShapeDtypeStruct((NW, NP), _f32),
            jax.ShapeDtypeStruct((NW, NP), _f32),
        ],
        mesh=_mesh(),
        compiler_params=pltpu.CompilerParams(needs_layout_passes=False),
        scratch_types=[
            pltpu.VMEM((2 * (HALF + NBUF) * CE,), _i32),
            pltpu.VMEM((NP,), _f32),
        ],
    )(dw, dr)


def _agg_body(table_hbm, sidx_hbm, didx_hbm, out_hbm,
              sidx_v, didx_v, rows_v, acc_sh, *sems):
    cid = lax.axis_index("c")
    sid = lax.axis_index("s")
    wid = cid * NS + sid
    gsem = sems[:NBUF]
    ssem = sems[NBUF:]
    zeros = jnp.zeros((LANES,), _f32)

    def z(i, _):
        r = i // (H // LANES)
        k = i % (H // LANES)
        rows_v[0, r, pl.ds(k * LANES, LANES)] = zeros
        return 0
    lax.fori_loop(0, (CE * H) // LANES, z, 0)
    for j in range(RSUB // CE):
        pltpu.sync_copy(rows_v.at[0],
                        acc_sh.at[pl.ds(sid * RSUB + j * CE, CE)])
    plsc.subcore_barrier()

    for h in range(2):
        pltpu.sync_copy(sidx_hbm.at[wid, h], sidx_v)
        pltpu.sync_copy(didx_hbm.at[wid, h], didx_v)

        for b in range(NBUF):
            pltpu.async_copy(table_hbm.at[sidx_v.at[b]], rows_v.at[b],
                             gsem[b])

        def quad(j, _):
            c = j * NBUF
            scat = []
            for b in range(NBUF):
                pltpu.make_async_copy(table_hbm.at[sidx_v.at[c + b]],
                                      rows_v.at[b], gsem[b]).wait()
                scat.append(pltpu.async_copy(
                    rows_v.at[b], acc_sh.at[didx_v.at[c + b]], ssem[b],
                    add=True))
            for b in range(NBUF):
                scat[b].wait()
                pltpu.async_copy(table_hbm.at[sidx_v.at[c + NBUF + b]],
                                 rows_v.at[b], gsem[b])
            return 0
        lax.fori_loop(0, HALF // NBUF, quad, 0)

        for b in range(NBUF):
            pltpu.make_async_copy(table_hbm.at[sidx_v.at[HALF + b]],
                                  rows_v.at[b], gsem[b]).wait()
    plsc.subcore_barrier()

    pltpu.sync_copy(acc_sh.at[pl.ds(sid * RSUB, RSUB)],
                    out_hbm.at[cid, pl.ds(sid * RSUB, RSUB)])


@jax.jit
def _sc_agg(table, sidx, didx):
    return pl.kernel(
        _agg_body,
        out_type=jax.ShapeDtypeStruct((NC, NP, H), _f32),
        mesh=_mesh(),
        compiler_params=pltpu.CompilerParams(needs_layout_passes=False),
        scratch_types=[
            pltpu.VMEM((HALF + NBUF, CE), _i32),
            pltpu.VMEM((HALF + NBUF, CE), _i32),
            pltpu.VMEM((NBUF, CE, H), _f32),
            pltpu.VMEM_SHARED((NP, H), _f32),
        ] + [pltpu.SemaphoreType.DMA] * (2 * NBUF),
    )(table, sidx, didx)


def _cls_body(ax_hbm, px_hbm, aidx_hbm, pidx_hbm, out_hbm,
              aidx_v, pidx_v, arows_v, prows_v, obuf_v, a0, a1, p0, p1):
    wid = _wid()
    asem = (a0, a1)
    psem = (p0, p1)
    zeros = jnp.zeros((LANES,), _f32)

    def z(i, _):
        obuf_v[pl.ds(i * LANES, LANES)] = zeros
        return 0
    lax.fori_loop(0, (CL * C) // LANES, z, 0)
    pltpu.sync_copy(aidx_hbm.at[wid], aidx_v)
    pltpu.sync_copy(pidx_hbm.at[wid], pidx_v)

    for b in range(2):
        pltpu.async_copy(ax_hbm.at[aidx_v.at[b]], arows_v.at[b], asem[b])
        pltpu.async_copy(px_hbm.at[pidx_v.at[b]], prows_v.at[b], psem[b])

    def pair(j, _):
        for b in range(2):
            c = 2 * j + b
            pltpu.make_async_copy(ax_hbm.at[aidx_v.at[c]],
                                  arows_v.at[b], asem[b]).wait()
            pltpu.make_async_copy(px_hbm.at[pidx_v.at[c]],
                                  prows_v.at[b], psem[b]).wait()

            def row(r, _):
                acc = (arows_v[b, r, pl.ds(0, LANES)]
                       * prows_v[b, r, pl.ds(0, LANES)])
                for k in range(1, H // LANES):
                    acc = acc + (arows_v[b, r, pl.ds(k * LANES, LANES)]
                                 * prows_v[b, r, pl.ds(k * LANES, LANES)])
                pos = jnp.full((LANES,), c * C + r, _i32)
                plsc.addupdate_scatter(obuf_v, [pos], acc)
                return 0
            lax.fori_loop(0, C, row, 0)

            pltpu.async_copy(ax_hbm.at[aidx_v.at[c + 2]],
                             arows_v.at[b], asem[b])
            pltpu.async_copy(px_hbm.at[pidx_v.at[c + 2]],
                             prows_v.at[b], psem[b])
        return 0
    lax.fori_loop(0, CL // 2, pair, 0)

    for b in range(2):
        pltpu.make_async_copy(ax_hbm.at[aidx_v.at[CL + b]],
                              arows_v.at[b], asem[b]).wait()
        pltpu.make_async_copy(px_hbm.at[pidx_v.at[CL + b]],
                              prows_v.at[b], psem[b]).wait()
    pltpu.sync_copy(obuf_v, out_hbm.at[wid])


@jax.jit
def _sc_cls(ax, px, aidx, pidx):
    return pl.kernel(
        _cls_body,
        out_type=jax.ShapeDtypeStruct((NW, CL * C), _f32),
        mesh=_mesh(),
        compiler_params=pltpu.CompilerParams(needs_layout_passes=False),
        scratch_types=[
            pltpu.VMEM((CL + 2, C), _i32),
            pltpu.VMEM((CL + 2, C), _i32),
            pltpu.VMEM((2, C, H), _f32),
            pltpu.VMEM((2, C, H), _f32),
            pltpu.VMEM((CL * C,), _f32),
        ] + [pltpu.SemaphoreType.DMA] * 4,
    )(ax, px, aidx, pidx)


def _tc1_body(cntw_ref, px_ref, wnb_ref, wroot_ref, ph1_ref, invw_ref):
    cnt = jnp.sum(cntw_ref[...][:, :N], axis=0)
    ind = (cnt > 0.0).astype(_f32)
    colsum = jnp.sum(wnb_ref[...], axis=0)
    ph1 = ind[:, None] * colsum[None, :] + jnp.dot(
        px_ref[...], wroot_ref[...], preferred_element_type=_f32)
    ph1_ref[...] = jnp.maximum(ph1, 0.0)
    invw_ref[...] = (1.0 / jnp.maximum(cnt, 1.0))[:, None]


@jax.jit
def _tc1(cntw, px, wnb, wroot):
    return pl.pallas_call(
        _tc1_body,
        out_shape=[
            jax.ShapeDtypeStruct((N, H), _f32),
            jax.ShapeDtypeStruct((N, 1), _f32),
        ],
    )(cntw, px, wnb, wroot)


def _tc2_body(aggr_ref, cntr_ref, wnb_ref, wroot_ref, ah1_ref, invr_ref):
    cnt = jnp.sum(cntr_ref[...][:, :N], axis=0)
    inv = 1.0 / jnp.maximum(cnt, 1.0)
    a = aggr_ref[...]
    mean = (a[0, :N, :] + a[1, :N, :]) * inv[:, None]
    colsum = jnp.sum(wroot_ref[...], axis=0)
    ah1 = jnp.dot(mean, wnb_ref[...], preferred_element_type=_f32) \
        + colsum[None, :]
    ah1_ref[...] = jnp.maximum(ah1, 0.0)
    invr_ref[...] = inv[:, None]


@jax.jit
def _tc2(aggr, cntr, wnb, wroot):
    return pl.pallas_call(
        _tc2_body,
        out_shape=[
            jax.ShapeDtypeStruct((N, H), _f32),
            jax.ShapeDtypeStruct((N, 1), _f32),
        ],
    )(aggr, cntr, wnb, wroot)


def _tc3_body(agg_ref, inv_ref, h1_ref, wnb_ref, wroot_ref, h2_ref):
    a = agg_ref[...]
    mean = (a[0, :N, :] + a[1, :N, :]) * inv_ref[...]
    h2_ref[...] = jnp.dot(mean, wnb_ref[...], preferred_element_type=_f32) \
        + jnp.dot(h1_ref[...], wroot_ref[...], preferred_element_type=_f32)


@jax.jit
def _tc3(agg, inv, h1, wnb, wroot):
    return pl.pallas_call(
        _tc3_body,
        out_shape=jax.ShapeDtypeStruct((N, H), _f32),
    )(agg, inv, h1, wnb, wroot)


def _pad_edges(x, fill):
    pad = NW * CW * CE - x.shape[0]
    g = jnp.concatenate([x, jnp.full((pad,), fill, _i32)]).reshape(
        NW, 2, HALF, CE)
    extra = jnp.full((NW, 2, NBUF, CE), fill, _i32)
    return jnp.concatenate([g, extra], axis=2)


def _pad_labels(x, fill):
    pad = NW * CL * C - x.shape[0]
    g = jnp.concatenate([x, jnp.full((pad,), fill, _i32)]).reshape(
        NW, CL, C)
    extra = jnp.full((NW, 2, C), fill, _i32)
    return jnp.concatenate([g, extra], axis=1)


def kernel(paper_x, edge_index_writes, edge_index_rev, edge_label_index,
           W1_writes_nb, W1_writes_root, W1_rev_nb, W1_rev_root,
           W2_writes_nb, W2_writes_root, W2_rev_nb, W2_rev_root):
    sw = jnp.asarray(edge_index_writes[0], _i32)
    dw = jnp.asarray(edge_index_writes[1], _i32)
    sr = jnp.asarray(edge_index_rev[0], _i32)
    dr = jnp.asarray(edge_index_rev[1], _i32)
    ali = jnp.asarray(edge_label_index[0], _i32)
    pli = jnp.asarray(edge_label_index[1], _i32)

    sw_p = _pad_edges(sw, 0)
    dw_p = _pad_edges(dw, N)
    sr_p = _pad_edges(sr, 0)
    dr_p = _pad_edges(dr, N)

    cntw_part, cntr_part = _sc_counts(dw_p.reshape(NW, -1),
                                      dr_p.reshape(NW, -1))
    ph1, invw = _tc1(cntw_part, paper_x, W1_writes_nb, W1_writes_root)
    aggr1 = _sc_agg(paper_x, sr_p, dr_p)
    ah1, invr = _tc2(aggr1, cntr_part, W1_rev_nb, W1_rev_root)
    aggr2 = _sc_agg(ph1, sr_p, dr_p)
    aggw2 = _sc_agg(ah1, sw_p, dw_p)
    ph2 = _tc3(aggw2, invw, ph1, W2_writes_nb, W2_writes_root)
    ah2 = _tc3(aggr2, invr, ah1, W2_rev_nb, W2_rev_root)

    ali_p = _pad_labels(ali, 0)
    pli_p = _pad_labels(pli, 0)
    out = _sc_cls(ah2, ph2, ali_p, pli_p)
    return out.reshape(-1)[:L]


# speedup vs baseline: 1.6525x; 1.6525x over previous
import functools

import jax
import jax.numpy as jnp
from jax import lax
from jax.experimental import pallas as pl
from jax.experimental.pallas import tpu as pltpu
from jax.experimental.pallas import tpu_sc as plsc

N = 10000
H = 128
E = 320000
L = 100000

NC = 2
NS = 16
NW = NC * NS
LANES = 16

C = 128
CE = 64
CW = 160
QTR = CW // 4
NBUF = 4
NP = 10112
RSUB = NP // NS
CL = 26

_f32 = jnp.float32
_i32 = jnp.int32


def _mesh():
    return plsc.VectorSubcoreMesh(core_axis_name="c", subcore_axis_name="s")


def _wid():
    return lax.axis_index("c") * NS + lax.axis_index("s")


def _counts_body(dw_hbm, dr_hbm, outw_hbm, outr_hbm, idx_v, cnt_v):
    wid = _wid()
    ones = jnp.ones((LANES,), _f32)
    zeros = jnp.zeros((LANES,), _f32)

    def one_type(d_hbm, out_hbm):
        def z(i, _):
            cnt_v[pl.ds(i * LANES, LANES)] = zeros
            return 0
        lax.fori_loop(0, NP // LANES, z, 0)
        pltpu.sync_copy(d_hbm.at[wid], idx_v)

        def upd(i, _):
            idx = idx_v[pl.ds(i * LANES, LANES)]
            plsc.addupdate_scatter(cnt_v, [idx], ones)
            return 0
        lax.fori_loop(0, (CW * CE) // LANES, upd, 0)
        pltpu.sync_copy(cnt_v, out_hbm.at[wid])

    one_type(dw_hbm, outw_hbm)
    one_type(dr_hbm, outr_hbm)


@jax.jit
def _sc_counts(dw, dr):
    return pl.kernel(
        _counts_body,
        out_type=[
            jax.ShapeDtypeStruct((NW, NP), _f32),
            jax.ShapeDtypeStruct((NW, NP), _f32),
        ],
        mesh=_mesh(),
        compiler_params=pltpu.CompilerParams(needs_layout_passes=False),
        scratch_types=[
            pltpu.VMEM((CW * CE,), _i32),
            pltpu.VMEM((NP,), _f32),
        ],
    )(dw, dr)


def _agg_body(table_hbm, sidx_hbm, didx_hbm, out_hbm,
              sidx_v, didx_v, rows_v, acc_sh, *sems):
    cid = lax.axis_index("c")
    sid = lax.axis_index("s")
    wid = cid * NS + sid
    gsem = sems[:NBUF]
    ssem = sems[NBUF:]
    zeros = jnp.zeros((LANES,), _f32)

    def z(i, _):
        r = i // (H // LANES)
        k = i % (H // LANES)
        rows_v[0, r, pl.ds(k * LANES, LANES)] = zeros
        return 0
    lax.fori_loop(0, (CE * H) // LANES, z, 0)
    for j in range(RSUB // CE):
        pltpu.sync_copy(rows_v.at[0],
                        acc_sh.at[pl.ds(sid * RSUB + j * CE, CE)])
    rem = RSUB % CE
    if rem:
        pltpu.sync_copy(rows_v.at[0, pl.ds(0, rem)],
                        acc_sh.at[pl.ds(sid * RSUB + (RSUB // CE) * CE, rem)])
    plsc.subcore_barrier()

    for h in range(4):
        pltpu.sync_copy(sidx_hbm.at[wid, h], sidx_v)
        pltpu.sync_copy(didx_hbm.at[wid, h], didx_v)

        def group(j, _):
            c = j * NBUF
            gd = [pltpu.async_copy(table_hbm.at[sidx_v.at[c + b]],
                                   rows_v.at[b], gsem[b])
                  for b in range(NBUF)]
            sd = []
            for b in range(NBUF):
                gd[b].wait()
                sd.append(pltpu.async_copy(
                    rows_v.at[b], acc_sh.at[didx_v.at[c + b]], ssem[b],
                    add=True))
            for b in range(NBUF):
                sd[b].wait()
            return 0
        lax.fori_loop(0, QTR // NBUF, group, 0)
    plsc.subcore_barrier()

    pltpu.sync_copy(acc_sh.at[pl.ds(sid * RSUB, RSUB)],
                    out_hbm.at[cid, pl.ds(sid * RSUB, RSUB)])


@jax.jit
def _sc_agg(table, sidx, didx):
    return pl.kernel(
        _agg_body,
        out_type=jax.ShapeDtypeStruct((NC, NP, H), _f32),
        mesh=_mesh(),
        compiler_params=pltpu.CompilerParams(needs_layout_passes=False),
        scratch_types=[
            pltpu.VMEM((QTR, CE), _i32),
            pltpu.VMEM((QTR, CE), _i32),
            pltpu.VMEM((NBUF, CE, H), _f32),
            pltpu.VMEM_SHARED((NP, H), _f32),
        ] + [pltpu.SemaphoreType.DMA] * (2 * NBUF),
    )(table, sidx, didx)


def _cls_body(ax_hbm, px_hbm, aidx_hbm, pidx_hbm, out_hbm,
              aidx_v, pidx_v, arows_v, prows_v, obuf_v, a0, a1, p0, p1):
    wid = _wid()
    asem = (a0, a1)
    psem = (p0, p1)
    zeros = jnp.zeros((LANES,), _f32)

    def z(i, _):
        obuf_v[pl.ds(i * LANES, LANES)] = zeros
        return 0
    lax.fori_loop(0, (CL * C) // LANES, z, 0)
    pltpu.sync_copy(aidx_hbm.at[wid], aidx_v)
    pltpu.sync_copy(pidx_hbm.at[wid], pidx_v)

    def pair(j, _):
        c0 = 2 * j
        ad = [pltpu.async_copy(ax_hbm.at[aidx_v.at[c0 + b]],
                               arows_v.at[b], asem[b]) for b in range(2)]
        pd = [pltpu.async_copy(px_hbm.at[pidx_v.at[c0 + b]],
                               prows_v.at[b], psem[b]) for b in range(2)]
        for b in range(2):
            c = c0 + b
            ad[b].wait()
            pd[b].wait()

            def row(r, _):
                acc = (arows_v[b, r, pl.ds(0, LANES)]
                       * prows_v[b, r, pl.ds(0, LANES)])
                for k in range(1, H // LANES):
                    acc = acc + (arows_v[b, r, pl.ds(k * LANES, LANES)]
                                 * prows_v[b, r, pl.ds(k * LANES, LANES)])
                pos = jnp.full((LANES,), c * C + r, _i32)
                plsc.addupdate_scatter(obuf_v, [pos], acc)
                return 0
            lax.fori_loop(0, C, row, 0)
        return 0
    lax.fori_loop(0, CL // 2, pair, 0)
    pltpu.sync_copy(obuf_v, out_hbm.at[wid])


@jax.jit
def _sc_cls(ax, px, aidx, pidx):
    return pl.kernel(
        _cls_body,
        out_type=jax.ShapeDtypeStruct((NW, CL * C), _f32),
        mesh=_mesh(),
        compiler_params=pltpu.CompilerParams(needs_layout_passes=False),
        scratch_types=[
            pltpu.VMEM((CL, C), _i32),
            pltpu.VMEM((CL, C), _i32),
            pltpu.VMEM((2, C, H), _f32),
            pltpu.VMEM((2, C, H), _f32),
            pltpu.VMEM((CL * C,), _f32),
        ] + [pltpu.SemaphoreType.DMA] * 4,
    )(ax, px, aidx, pidx)


def _tc1_body(cntw_ref, px_ref, wnb_ref, wroot_ref, ph1_ref, invw_ref):
    cnt = jnp.sum(cntw_ref[...][:, :N], axis=0)
    ind = (cnt > 0.0).astype(_f32)
    colsum = jnp.sum(wnb_ref[...], axis=0)
    ph1 = ind[:, None] * colsum[None, :] + jnp.dot(
        px_ref[...], wroot_ref[...], preferred_element_type=_f32)
    ph1_ref[...] = jnp.maximum(ph1, 0.0)
    invw_ref[...] = (1.0 / jnp.maximum(cnt, 1.0))[:, None]


@jax.jit
def _tc1(cntw, px, wnb, wroot):
    return pl.pallas_call(
        _tc1_body,
        out_shape=[
            jax.ShapeDtypeStruct((N, H), _f32),
            jax.ShapeDtypeStruct((N, 1), _f32),
        ],
    )(cntw, px, wnb, wroot)


def _tc2_body(aggr_ref, cntr_ref, wnb_ref, wroot_ref, ah1_ref, invr_ref):
    cnt = jnp.sum(cntr_ref[...][:, :N], axis=0)
    inv = 1.0 / jnp.maximum(cnt, 1.0)
    a = aggr_ref[...]
    mean = (a[0, :N, :] + a[1, :N, :]) * inv[:, None]
    colsum = jnp.sum(wroot_ref[...], axis=0)
    ah1 = jnp.dot(mean, wnb_ref[...], preferred_element_type=_f32) \
        + colsum[None, :]
    ah1_ref[...] = jnp.maximum(ah1, 0.0)
    invr_ref[...] = inv[:, None]


@jax.jit
def _tc2(aggr, cntr, wnb, wroot):
    return pl.pallas_call(
        _tc2_body,
        out_shape=[
            jax.ShapeDtypeStruct((N, H), _f32),
            jax.ShapeDtypeStruct((N, 1), _f32),
        ],
    )(aggr, cntr, wnb, wroot)


def _tc3_body(agg_ref, inv_ref, h1_ref, wnb_ref, wroot_ref, h2_ref):
    a = agg_ref[...]
    mean = (a[0, :N, :] + a[1, :N, :]) * inv_ref[...]
    h2_ref[...] = jnp.dot(mean, wnb_ref[...], preferred_element_type=_f32) \
        + jnp.dot(h1_ref[...], wroot_ref[...], preferred_element_type=_f32)


@jax.jit
def _tc3(agg, inv, h1, wnb, wroot):
    return pl.pallas_call(
        _tc3_body,
        out_shape=jax.ShapeDtypeStruct((N, H), _f32),
    )(agg, inv, h1, wnb, wroot)


def _pad_edges(x, fill):
    pad = NW * CW * CE - x.shape[0]
    return jnp.concatenate([x, jnp.full((pad,), fill, _i32)]).reshape(
        NW, 4, QTR, CE)


def _pad_labels(x, fill):
    pad = NW * CL * C - x.shape[0]
    return jnp.concatenate([x, jnp.full((pad,), fill, _i32)]).reshape(
        NW, CL, C)


def kernel(paper_x, edge_index_writes, edge_index_rev, edge_label_index,
           W1_writes_nb, W1_writes_root, W1_rev_nb, W1_rev_root,
           W2_writes_nb, W2_writes_root, W2_rev_nb, W2_rev_root):
    sw = jnp.asarray(edge_index_writes[0], _i32)
    dw = jnp.asarray(edge_index_writes[1], _i32)
    sr = jnp.asarray(edge_index_rev[0], _i32)
    dr = jnp.asarray(edge_index_rev[1], _i32)
    ali = jnp.asarray(edge_label_index[0], _i32)
    pli = jnp.asarray(edge_label_index[1], _i32)

    sw_p = _pad_edges(sw, 0)
    dw_p = _pad_edges(dw, N)
    sr_p = _pad_edges(sr, 0)
    dr_p = _pad_edges(dr, N)

    cntw_part, cntr_part = _sc_counts(dw_p.reshape(NW, -1),
                                      dr_p.reshape(NW, -1))
    ph1, invw = _tc1(cntw_part, paper_x, W1_writes_nb, W1_writes_root)
    aggr1 = _sc_agg(paper_x, sr_p, dr_p)
    ah1, invr = _tc2(aggr1, cntr_part, W1_rev_nb, W1_rev_root)
    aggr2 = _sc_agg(ph1, sr_p, dr_p)
    aggw2 = _sc_agg(ah1, sw_p, dw_p)
    ph2 = _tc3(aggw2, invw, ph1, W2_writes_nb, W2_writes_root)
    ah2 = _tc3(aggr2, invr, ah1, W2_rev_nb, W2_rev_root)

    ali_p = _pad_labels(ali, 0)
    pli_p = _pad_labels(pli, 0)
    out = _sc_cls(ah2, ph2, ali_p, pli_p)
    return out.reshape(-1)[:L]


# speedup vs baseline: 4.7692x; 2.8860x over previous
import functools

import jax
import jax.numpy as jnp
from jax import lax
from jax.experimental import pallas as pl
from jax.experimental.pallas import tpu as pltpu
from jax.experimental.pallas import tpu_sc as plsc

N = 10000
H = 128
E = 320000
L = 100000

NC = 2
NS = 16
NW = NC * NS
LANES = 16

C = 128
CE = 64
CW = 160
QTR = CW // 4
NBUF = 4
NP = 10112
RSUB = NP // NS
CL = 26

_f32 = jnp.float32
_i32 = jnp.int32


def _mesh():
    return plsc.VectorSubcoreMesh(core_axis_name="c", subcore_axis_name="s")


def _wid():
    return lax.axis_index("c") * NS + lax.axis_index("s")


def _counts_body(dw_hbm, dr_hbm, outw_hbm, outr_hbm, idx_v, cnt_v):
    wid = _wid()
    ones = jnp.ones((LANES,), _f32)
    zeros = jnp.zeros((LANES,), _f32)

    def one_type(d_hbm, out_hbm):
        def z(i, _):
            cnt_v[pl.ds(i * LANES, LANES)] = zeros
            return 0
        lax.fori_loop(0, NP // LANES, z, 0)
        pltpu.sync_copy(d_hbm.at[wid], idx_v)

        def upd(i, _):
            idx = idx_v[pl.ds(i * LANES, LANES)]
            plsc.addupdate_scatter(cnt_v, [idx], ones)
            return 0
        lax.fori_loop(0, (CW * CE) // LANES, upd, 0)
        pltpu.sync_copy(cnt_v, out_hbm.at[wid])

    one_type(dw_hbm, outw_hbm)
    one_type(dr_hbm, outr_hbm)


@jax.jit
def _sc_counts(dw, dr):
    return pl.kernel(
        _counts_body,
        out_type=[
            jax.ShapeDtypeStruct((NW, NP), _f32),
            jax.ShapeDtypeStruct((NW, NP), _f32),
        ],
        mesh=_mesh(),
        compiler_params=pltpu.CompilerParams(needs_layout_passes=False),
        scratch_types=[
            pltpu.VMEM((CW * CE,), _i32),
            pltpu.VMEM((NP,), _f32),
        ],
    )(dw, dr)


def _agg_body(table_hbm, sidx_hbm, didx_hbm, out_hbm,
              sidx_v, didx_v, rows_v, acc_sh, *sems):
    cid = lax.axis_index("c")
    sid = lax.axis_index("s")
    wid = cid * NS + sid
    gsem = sems[:NBUF]
    ssem = sems[NBUF:]
    zeros = jnp.zeros((LANES,), _f32)

    def z(i, _):
        r = i // (H // LANES)
        k = i % (H // LANES)
        rows_v[0, r, pl.ds(k * LANES, LANES)] = zeros
        return 0
    lax.fori_loop(0, (CE * H) // LANES, z, 0)
    for j in range(RSUB // CE):
        pltpu.sync_copy(rows_v.at[0],
                        acc_sh.at[pl.ds(sid * RSUB + j * CE, CE)])
    rem = RSUB % CE
    if rem:
        pltpu.sync_copy(rows_v.at[0, pl.ds(0, rem)],
                        acc_sh.at[pl.ds(sid * RSUB + (RSUB // CE) * CE, rem)])
    plsc.subcore_barrier()

    for h in range(4):
        pltpu.sync_copy(sidx_hbm.at[wid, h], sidx_v)
        pltpu.sync_copy(didx_hbm.at[wid, h], didx_v)

        def group(j, _):
            c = j * NBUF
            gd = [pltpu.async_copy(table_hbm.at[sidx_v.at[c + b]],
                                   rows_v.at[b], gsem[b])
                  for b in range(NBUF)]
            sd = []
            for b in range(NBUF):
                gd[b].wait()
                sd.append(pltpu.async_copy(
                    rows_v.at[b], acc_sh.at[didx_v.at[c + b]], ssem[b],
                    add=True))
            for b in range(NBUF):
                sd[b].wait()
            return 0
        lax.fori_loop(0, QTR // NBUF, group, 0)
    plsc.subcore_barrier()

    pltpu.sync_copy(acc_sh.at[pl.ds(sid * RSUB, RSUB)],
                    out_hbm.at[cid, pl.ds(sid * RSUB, RSUB)])


@jax.jit
def _sc_agg(table, sidx, didx):
    return pl.kernel(
        _agg_body,
        out_type=jax.ShapeDtypeStruct((NC, NP, H), _f32),
        mesh=_mesh(),
        compiler_params=pltpu.CompilerParams(needs_layout_passes=False),
        scratch_types=[
            pltpu.VMEM((QTR, CE), _i32),
            pltpu.VMEM((QTR, CE), _i32),
            pltpu.VMEM((NBUF, CE, H), _f32),
            pltpu.VMEM_SHARED((NP, H), _f32),
        ] + [pltpu.SemaphoreType.DMA] * (2 * NBUF),
    )(table, sidx, didx)


def _cls_body(ax_hbm, px_hbm, aidx_hbm, pidx_hbm, out_hbm,
              aidx_v, pidx_v, arows_v, prows_v, obuf_v, a0, a1, p0, p1):
    wid = _wid()
    asem = (a0, a1)
    psem = (p0, p1)
    zeros = jnp.zeros((LANES,), _f32)

    def z(i, _):
        obuf_v[pl.ds(i * LANES, LANES)] = zeros
        return 0
    lax.fori_loop(0, (CL * C) // LANES, z, 0)
    pltpu.sync_copy(aidx_hbm.at[wid], aidx_v)
    pltpu.sync_copy(pidx_hbm.at[wid], pidx_v)

    def pair(j, _):
        c0 = 2 * j
        ad = [pltpu.async_copy(ax_hbm.at[aidx_v.at[c0 + b]],
                               arows_v.at[b], asem[b]) for b in range(2)]
        pd = [pltpu.async_copy(px_hbm.at[pidx_v.at[c0 + b]],
                               prows_v.at[b], psem[b]) for b in range(2)]
        for b in range(2):
            c = c0 + b
            ad[b].wait()
            pd[b].wait()

            def row(r, _):
                acc = (arows_v[b, r, pl.ds(0, LANES)]
                       * prows_v[b, r, pl.ds(0, LANES)])
                for k in range(1, H // LANES):
                    acc = acc + (arows_v[b, r, pl.ds(k * LANES, LANES)]
                                 * prows_v[b, r, pl.ds(k * LANES, LANES)])
                pos = jnp.full((LANES,), c * C + r, _i32)
                plsc.addupdate_scatter(obuf_v, [pos], acc)
                return 0
            lax.fori_loop(0, C, row, 0)
        return 0
    lax.fori_loop(0, CL // 2, pair, 0)
    pltpu.sync_copy(obuf_v, out_hbm.at[wid])


@jax.jit
def _sc_cls(ax, px, aidx, pidx):
    return pl.kernel(
        _cls_body,
        out_type=jax.ShapeDtypeStruct((NW, CL * C), _f32),
        mesh=_mesh(),
        compiler_params=pltpu.CompilerParams(needs_layout_passes=False),
        scratch_types=[
            pltpu.VMEM((CL, C), _i32),
            pltpu.VMEM((CL, C), _i32),
            pltpu.VMEM((2, C, H), _f32),
            pltpu.VMEM((2, C, H), _f32),
            pltpu.VMEM((CL * C,), _f32),
        ] + [pltpu.SemaphoreType.DMA] * 4,
    )(ax, px, aidx, pidx)


def _tc1_body(cntw_ref, px_ref, wnb_ref, wroot_ref, ph1_ref, invw_ref):
    cnt = jnp.sum(cntw_ref[...][:, :N], axis=0)
    ind = (cnt > 0.0).astype(_f32)
    colsum = jnp.sum(wnb_ref[...], axis=0)
    ph1 = ind[:, None] * colsum[None, :] + jnp.dot(
        px_ref[...], wroot_ref[...], preferred_element_type=_f32)
    ph1_ref[...] = jnp.maximum(ph1, 0.0)
    invw_ref[...] = (1.0 / jnp.maximum(cnt, 1.0))[:, None]


@jax.jit
def _tc1(cntw, px, wnb, wroot):
    return pl.pallas_call(
        _tc1_body,
        out_shape=[
            jax.ShapeDtypeStruct((N, H), _f32),
            jax.ShapeDtypeStruct((N, 1), _f32),
        ],
    )(cntw, px, wnb, wroot)


def _tc2_body(aggr_ref, cntr_ref, wnb_ref, wroot_ref, ah1_ref, invr_ref):
    cnt = jnp.sum(cntr_ref[...][:, :N], axis=0)
    inv = 1.0 / jnp.maximum(cnt, 1.0)
    a = aggr_ref[...]
    mean = (a[0, :N, :] + a[1, :N, :]) * inv[:, None]
    colsum = jnp.sum(wroot_ref[...], axis=0)
    ah1 = jnp.dot(mean, wnb_ref[...], preferred_element_type=_f32) \
        + colsum[None, :]
    ah1_ref[...] = jnp.maximum(ah1, 0.0)
    invr_ref[...] = inv[:, None]


@jax.jit
def _tc2(aggr, cntr, wnb, wroot):
    return pl.pallas_call(
        _tc2_body,
        out_shape=[
            jax.ShapeDtypeStruct((N, H), _f32),
            jax.ShapeDtypeStruct((N, 1), _f32),
        ],
    )(aggr, cntr, wnb, wroot)


def _tc3_body(agg_ref, inv_ref, h1_ref, wnb_ref, wroot_ref, h2_ref):
    a = agg_ref[...]
    mean = (a[0, :N, :] + a[1, :N, :]) * inv_ref[...]
    h2_ref[...] = jnp.dot(mean, wnb_ref[...], preferred_element_type=_f32) \
        + jnp.dot(h1_ref[...], wroot_ref[...], preferred_element_type=_f32)


@jax.jit
def _tc3(agg, inv, h1, wnb, wroot):
    return pl.pallas_call(
        _tc3_body,
        out_shape=jax.ShapeDtypeStruct((N, H), _f32),
    )(agg, inv, h1, wnb, wroot)


def _pad_edges(x, trash):
    pad = NW * CW * CE - x.shape[0]
    if trash:
        fill = N + (jnp.arange(pad, dtype=_i32) % (NP - N))
    else:
        fill = jnp.arange(pad, dtype=_i32) % N
    return jnp.concatenate([x, fill]).reshape(NW, 4, QTR, CE)


def _pad_labels(x):
    pad = NW * CL * C - x.shape[0]
    fill = jnp.arange(pad, dtype=_i32) % N
    return jnp.concatenate([x, fill]).reshape(NW, CL, C)


def kernel(paper_x, edge_index_writes, edge_index_rev, edge_label_index,
           W1_writes_nb, W1_writes_root, W1_rev_nb, W1_rev_root,
           W2_writes_nb, W2_writes_root, W2_rev_nb, W2_rev_root):
    sw = jnp.asarray(edge_index_writes[0], _i32)
    dw = jnp.asarray(edge_index_writes[1], _i32)
    sr = jnp.asarray(edge_index_rev[0], _i32)
    dr = jnp.asarray(edge_index_rev[1], _i32)
    ali = jnp.asarray(edge_label_index[0], _i32)
    pli = jnp.asarray(edge_label_index[1], _i32)

    sw_p = _pad_edges(sw, trash=False)
    dw_p = _pad_edges(dw, trash=True)
    sr_p = _pad_edges(sr, trash=False)
    dr_p = _pad_edges(dr, trash=True)

    cntw_part, cntr_part = _sc_counts(dw_p.reshape(NW, -1),
                                      dr_p.reshape(NW, -1))
    ph1, invw = _tc1(cntw_part, paper_x, W1_writes_nb, W1_writes_root)
    aggr1 = _sc_agg(paper_x, sr_p, dr_p)
    ah1, invr = _tc2(aggr1, cntr_part, W1_rev_nb, W1_rev_root)
    aggr2 = _sc_agg(ph1, sr_p, dr_p)
    aggw2 = _sc_agg(ah1, sw_p, dw_p)
    ph2 = _tc3(aggw2, invw, ph1, W2_writes_nb, W2_writes_root)
    ah2 = _tc3(aggr2, invr, ah1, W2_rev_nb, W2_rev_root)

    ali_p = _pad_labels(ali)
    pli_p = _pad_labels(pli)
    out = _sc_cls(ah2, ph2, ali_p, pli_p)
    return out.reshape(-1)[:L]
